# baseline (speedup 1.0000x reference)
import functools

import jax
import jax.numpy as jnp
from jax import lax
from jax.experimental import pallas as pl
from jax.experimental.pallas import tpu as pltpu
from jax.experimental.pallas import tpu_sc as plsc
from jax._src.pallas import mpmd as _mpmd

_B = 32
_NW = 32


def _sc_mesh():
    return plsc.VectorSubcoreMesh(core_axis_name="c", subcore_axis_name="s")


def _worker_id():
    return lax.axis_index("s") * 2 + lax.axis_index("c")


def _gather_rows(table, idx):
    _, D = table.shape
    N = idx.shape[0]
    per_w = N // _NW

    @functools.partial(
        pl.kernel,
        out_type=jax.ShapeDtypeStruct((N, D), table.dtype),
        mesh=_sc_mesh(),
        scratch_types=[
            pltpu.VMEM((per_w,), jnp.int32),
            pltpu.VMEM((per_w, D), jnp.float32),
            pltpu.SemaphoreType.DMA,
        ],
        compiler_params=pltpu.CompilerParams(use_tc_tiling_on_sc=False),
    )
    def k(table_hbm, idx_hbm, out_hbm, idx_v, rows_v, sem):
        base = _worker_id() * per_w
        pltpu.sync_copy(idx_hbm.at[pl.ds(base, per_w)], idx_v)
        pltpu.async_copy(table_hbm.at[idx_v], rows_v, sem).wait()
        pltpu.sync_copy(rows_v, out_hbm.at[pl.ds(base, per_w)])

    return k(table, idx)


def _scatter_map(zmap, idx2, val2):
    per_w = idx2.shape[1]

    def body(z_in, idx_hbm, val_hbm, z_out, idx_v, val_v, sem):
        del z_in
        wid = _worker_id()
        pltpu.sync_copy(idx_hbm.at[wid], idx_v)
        pltpu.sync_copy(val_hbm.at[wid], val_v)
        pltpu.async_copy(val_v, z_out.at[idx_v], sem).wait()

    fn = _mpmd._mpmd_map(
        [(_sc_mesh(), body)],
        out_types=jax.ShapeDtypeStruct(zmap.shape, zmap.dtype),
        input_output_aliases={0: 0},
        scratch_types=[
            pltpu.VMEM((per_w,), jnp.int32),
            pltpu.VMEM((per_w,), jnp.int32),
            pltpu.SemaphoreType.DMA,
        ],
        compiler_params=pltpu.CompilerParams(use_tc_tiling_on_sc=False),
    )
    return fn(zmap, idx2, val2)


def _prologue(occT, w_sent, b_sent, bi_row, mi_row, bi_col, mi_col,
              weg, be_row, M):
    U, L, NM = occT.shape
    H = weg.shape[1]
    B = _B

    def body(occ_ref, w_ref, b_ref, bi_ref, mi_ref, bic_ref, mic_ref,
             weg_ref, be_ref, hs_ref, zval_ref, zidx_ref):
        acc_logit = jnp.zeros((L, NM), jnp.float32)
        acc_sum = jnp.zeros((L, NM), jnp.float32)
        for u in range(U):
            s = occ_ref[u]
            acc_logit = acc_logit + s * w_ref[u]
            acc_sum = acc_sum + s
        likes = jax.nn.sigmoid(acc_logit + b_ref[0])
        mentioned = (acc_sum > 0).astype(jnp.float32)
        tri = (lax.broadcasted_iota(jnp.int32, (L, L), 1)
               <= lax.broadcasted_iota(jnp.int32, (L, L), 0)).astype(jnp.float32)
        cum = jnp.dot(tri, mentioned, preferred_element_type=jnp.float32)
        mask = (cum > 0.5).astype(jnp.float32)
        likes_m = likes * mask

        bi = bi_ref[:, :]
        mi = mi_ref[:, :]
        oh = (bi == lax.broadcasted_iota(jnp.int32, (B, NM), 0)
              ).astype(jnp.float32)
        a3 = likes_m[:, None, :] * oh[None, :, :]
        A = a3.reshape(L * B, NM)
        h = jnp.dot(A, weg_ref[:, :], preferred_element_type=jnp.float32)
        hs_ref[:, :] = jax.nn.sigmoid(h + be_ref[:, :])

        Pm = ((bic_ref[:, :] == bi)
              & (mic_ref[:, :] == mi)).astype(jnp.float32)
        ones_col = jnp.zeros((L, 1), jnp.float32) + 1.0
        s_col = lax.dot_general(mask, ones_col, (((0,), (0,)), ((), ())),
                                preferred_element_type=jnp.float32)
        zmax = jnp.max(Pm * s_col, axis=0, keepdims=True)
        zval_ref[:, :] = zmax.astype(jnp.int32)
        zidx_ref[:, :] = bi * M + mi

    return pl.pallas_call(
        body,
        out_shape=[
            jax.ShapeDtypeStruct((L * B, H), jnp.float32),
            jax.ShapeDtypeStruct((1, NM), jnp.int32),
            jax.ShapeDtypeStruct((1, NM), jnp.int32),
        ],
        in_specs=[
            pl.BlockSpec(memory_space=pltpu.VMEM),
            pl.BlockSpec(memory_space=pltpu.SMEM),
            pl.BlockSpec(memory_space=pltpu.SMEM),
            pl.BlockSpec(memory_space=pltpu.VMEM),
            pl.BlockSpec(memory_space=pltpu.VMEM),
            pl.BlockSpec(memory_space=pltpu.VMEM),
            pl.BlockSpec(memory_space=pltpu.VMEM),
            pl.BlockSpec(memory_space=pltpu.VMEM),
            pl.BlockSpec(memory_space=pltpu.VMEM),
        ],
    )(occT, w_sent, b_sent, bi_row, mi_row, bi_col, mi_col, weg, be_row)


def _decoder_matmul(hs3, Wd, bd_row, z2d):
    B, L, H = hs3.shape
    M = Wd.shape[1]
    TB = 32
    TM = 5120
    grid_m = (M + TM - 1) // TM

    def body(hs_ref, wd_ref, bd_ref, z_ref, out_ref):
        wd = wd_ref[:, :]
        bdv = bd_ref[:, :]
        lio = lax.broadcasted_iota(jnp.int32, (L, TM), 0)
        for bb in range(TB):
            mm = jnp.dot(hs_ref[bb], wd,
                         preferred_element_type=jnp.float32)
            keep = ((lio + z_ref[bb:bb + 1, :]) < L).astype(jnp.float32)
            out_ref[bb] = (mm + bdv) * keep

    return pl.pallas_call(
        body,
        grid=(grid_m, B // TB),
        in_specs=[
            pl.BlockSpec((TB, L, H), lambda i, b: (b, 0, 0)),
            pl.BlockSpec((H, TM), lambda i, b: (0, i)),
            pl.BlockSpec((1, TM), lambda i, b: (0, i)),
            pl.BlockSpec((TB, TM), lambda i, b: (b, i)),
        ],
        out_specs=pl.BlockSpec((TB, L, TM), lambda i, b: (b, 0, i)),
        out_shape=jax.ShapeDtypeStruct((B, L, M), jnp.float32),
        compiler_params=pltpu.CompilerParams(
            dimension_semantics=("arbitrary", "arbitrary"),
        ),
    )(hs3, Wd, bd_row, z2d)


def kernel(movie_occurrences, batch_indices, movie_ids, w_sent, b_sent,
           We, be, Wd, bd):
    NM, L, U = movie_occurrences.shape
    M, H = We.shape
    B = _B

    occT = jnp.transpose(movie_occurrences, (2, 1, 0))
    bi_row = batch_indices.reshape(1, NM)
    mi_row = movie_ids.reshape(1, NM)

    weg = _gather_rows(We, movie_ids)
    hs_lb, zval, zidx = _prologue(
        occT, w_sent, b_sent, bi_row, mi_row,
        batch_indices.reshape(NM, 1), movie_ids.reshape(NM, 1),
        weg, be.reshape(1, H), M)
    hs3 = hs_lb.reshape(L, B, H).transpose(1, 0, 2)

    z1d = _scatter_map(jnp.zeros((B * M,), jnp.int32),
                       zidx.reshape(_NW, NM // _NW),
                       zval.reshape(_NW, NM // _NW))
    z2d = z1d.reshape(B, M)

    return _decoder_matmul(hs3, Wd, bd.reshape(1, M), z2d)


# speedup vs baseline: 1.0105x; 1.0105x over previous
import functools

import jax
import jax.numpy as jnp
from jax import lax
from jax.experimental import pallas as pl
from jax.experimental.pallas import tpu as pltpu
from jax.experimental.pallas import tpu_sc as plsc
from jax._src.pallas import mpmd as _mpmd

_B = 32
_NW = 32


def _sc_mesh():
    return plsc.VectorSubcoreMesh(core_axis_name="c", subcore_axis_name="s")


def _worker_id():
    return lax.axis_index("s") * 2 + lax.axis_index("c")


def _sc_gather_scatter(We, gidx, zmap, zidx2, zval2):
    _, D = We.shape
    N = gidx.shape[0]
    per_w = N // _NW
    per_s = zidx2.shape[1]

    def body(table_hbm, gidx_hbm, z_in, zidx_hbm, zval_hbm,
             weg_out, z_out, idx_v, rows_v, zidx_v, zval_v, sem):
        del z_in
        wid = _worker_id()
        base = wid * per_w
        pltpu.sync_copy(gidx_hbm.at[pl.ds(base, per_w)], idx_v)
        pltpu.async_copy(table_hbm.at[idx_v], rows_v, sem).wait()
        pltpu.sync_copy(rows_v, weg_out.at[pl.ds(base, per_w)])
        pltpu.sync_copy(zidx_hbm.at[wid], zidx_v)
        pltpu.sync_copy(zval_hbm.at[wid], zval_v)
        pltpu.async_copy(zval_v, z_out.at[zidx_v], sem).wait()

    fn = _mpmd._mpmd_map(
        [(_sc_mesh(), body)],
        out_types=[
            jax.ShapeDtypeStruct((N, D), jnp.float32),
            jax.ShapeDtypeStruct(zmap.shape, zmap.dtype),
        ],
        input_output_aliases={2: 1},
        scratch_types=[
            pltpu.VMEM((per_w,), jnp.int32),
            pltpu.VMEM((per_w, D), jnp.float32),
            pltpu.VMEM((per_s,), jnp.int32),
            pltpu.VMEM((per_s,), jnp.int32),
            pltpu.SemaphoreType.DMA,
        ],
        compiler_params=pltpu.CompilerParams(use_tc_tiling_on_sc=False),
    )
    return fn(We, gidx, zmap, zidx2, zval2)


def _prologue(occT, w_sent, b_sent, bi_row, mi_row, bi_col, mi_col, M):
    U, L, NM = occT.shape
    B = _B

    def body(occ_ref, w_ref, b_ref, bi_ref, mi_ref, bic_ref, mic_ref,
             a_ref, zval_ref, zidx_ref):
        acc_logit = jnp.zeros((L, NM), jnp.float32)
        acc_sum = jnp.zeros((L, NM), jnp.float32)
        for u in range(U):
            s = occ_ref[u]
            acc_logit = acc_logit + s * w_ref[u]
            acc_sum = acc_sum + s
        likes = jax.nn.sigmoid(acc_logit + b_ref[0])
        mentioned = (acc_sum > 0).astype(jnp.float32)
        tri = (lax.broadcasted_iota(jnp.int32, (L, L), 1)
               <= lax.broadcasted_iota(jnp.int32, (L, L), 0)).astype(jnp.float32)
        cum = jnp.dot(tri, mentioned, preferred_element_type=jnp.float32)
        mask = (cum > 0.5).astype(jnp.float32)
        likes_m = likes * mask

        bi = bi_ref[:, :]
        mi = mi_ref[:, :]
        oh = (bi == lax.broadcasted_iota(jnp.int32, (B, NM), 0)
              ).astype(jnp.float32)
        a3 = likes_m[:, None, :] * oh[None, :, :]
        a_ref[:, :] = a3.reshape(L * B, NM)

        Pm = ((bic_ref[:, :] == bi)
              & (mic_ref[:, :] == mi)).astype(jnp.float32)
        ones_col = jnp.zeros((L, 1), jnp.float32) + 1.0
        s_col = lax.dot_general(mask, ones_col, (((0,), (0,)), ((), ())),
                                preferred_element_type=jnp.float32)
        zmax = jnp.max(Pm * s_col, axis=0, keepdims=True)
        zval_ref[:, :] = zmax.astype(jnp.int32)
        zidx_ref[:, :] = bi * M + mi

    return pl.pallas_call(
        body,
        out_shape=[
            jax.ShapeDtypeStruct((L * B, NM), jnp.float32),
            jax.ShapeDtypeStruct((1, NM), jnp.int32),
            jax.ShapeDtypeStruct((1, NM), jnp.int32),
        ],
        in_specs=[
            pl.BlockSpec(memory_space=pltpu.VMEM),
            pl.BlockSpec(memory_space=pltpu.SMEM),
            pl.BlockSpec(memory_space=pltpu.SMEM),
            pl.BlockSpec(memory_space=pltpu.VMEM),
            pl.BlockSpec(memory_space=pltpu.VMEM),
            pl.BlockSpec(memory_space=pltpu.VMEM),
            pl.BlockSpec(memory_space=pltpu.VMEM),
        ],
    )(occT, w_sent, b_sent, bi_row, mi_row, bi_col, mi_col)


def _encode(A, weg, be_row):
    LB, NM = A.shape
    H = weg.shape[1]

    def body(a_ref, weg_ref, be_ref, h_ref):
        h = jnp.dot(a_ref[:, :], weg_ref[:, :],
                    preferred_element_type=jnp.float32)
        h_ref[:, :] = jax.nn.sigmoid(h + be_ref[:, :])

    return pl.pallas_call(
        body,
        out_shape=jax.ShapeDtypeStruct((LB, H), jnp.float32),
    )(A, weg, be_row)


def _decoder_matmul(hs3, Wd, bd_row, z2d):
    B, L, H = hs3.shape
    M = Wd.shape[1]
    TB = 32
    TM = 5120
    grid_m = (M + TM - 1) // TM

    def body(hs_ref, wd_ref, bd_ref, z_ref, out_ref):
        wd = wd_ref[:, :]
        bdv = bd_ref[:, :]
        lio = lax.broadcasted_iota(jnp.int32, (L, TM), 0)
        for bb in range(TB):
            mm = jnp.dot(hs_ref[bb], wd,
                         preferred_element_type=jnp.float32)
            keep = ((lio + z_ref[bb:bb + 1, :]) < L).astype(jnp.float32)
            out_ref[bb] = (mm + bdv) * keep

    return pl.pallas_call(
        body,
        grid=(grid_m, B // TB),
        in_specs=[
            pl.BlockSpec((TB, L, H), lambda i, b: (b, 0, 0)),
            pl.BlockSpec((H, TM), lambda i, b: (0, i)),
            pl.BlockSpec((1, TM), lambda i, b: (0, i)),
            pl.BlockSpec((TB, TM), lambda i, b: (b, i)),
        ],
        out_specs=pl.BlockSpec((TB, L, TM), lambda i, b: (b, 0, i)),
        out_shape=jax.ShapeDtypeStruct((B, L, M), jnp.float32),
        compiler_params=pltpu.CompilerParams(
            dimension_semantics=("arbitrary", "arbitrary"),
        ),
    )(hs3, Wd, bd_row, z2d)


def kernel(movie_occurrences, batch_indices, movie_ids, w_sent, b_sent,
           We, be, Wd, bd):
    NM, L, U = movie_occurrences.shape
    M, H = We.shape
    B = _B

    occT = jnp.transpose(movie_occurrences, (2, 1, 0))
    bi_row = batch_indices.reshape(1, NM)
    mi_row = movie_ids.reshape(1, NM)

    A, zval, zidx = _prologue(
        occT, w_sent, b_sent, bi_row, mi_row,
        batch_indices.reshape(NM, 1), movie_ids.reshape(NM, 1), M)

    weg, z1d = _sc_gather_scatter(
        We, movie_ids, jnp.zeros((B * M,), jnp.int32),
        zidx.reshape(_NW, NM // _NW), zval.reshape(_NW, NM // _NW))

    h_lb = _encode(A, weg, be.reshape(1, H))
    hs3 = h_lb.reshape(L, B, H).transpose(1, 0, 2)
    z2d = z1d.reshape(B, M)

    return _decoder_matmul(hs3, Wd, bd.reshape(1, M), z2d)


# speedup vs baseline: 1.0109x; 1.0004x over previous
import functools

import jax
import jax.numpy as jnp
from jax import lax
from jax.experimental import pallas as pl
from jax.experimental.pallas import tpu as pltpu
from jax.experimental.pallas import tpu_sc as plsc
from jax._src.pallas import mpmd as _mpmd

_B = 32
_NW = 32


def _sc_mesh():
    return plsc.VectorSubcoreMesh(core_axis_name="c", subcore_axis_name="s")


def _worker_id():
    return lax.axis_index("s") * 2 + lax.axis_index("c")


def _sc_gather_scatter(We, gidx, zmap, zidx2, zval2):
    _, D = We.shape
    N = gidx.shape[0]
    per_w = N // _NW
    per_s = zidx2.shape[1]

    def body(table_hbm, gidx_hbm, z_in, zidx_hbm, zval_hbm,
             weg_out, z_out, idx_v, rows_v, zidx_v, zval_v, sem):
        del z_in
        wid = _worker_id()
        base = wid * per_w
        pltpu.sync_copy(gidx_hbm.at[pl.ds(base, per_w)], idx_v)
        pltpu.async_copy(table_hbm.at[idx_v], rows_v, sem).wait()
        pltpu.sync_copy(rows_v, weg_out.at[pl.ds(base, per_w)])
        pltpu.sync_copy(zidx_hbm.at[wid], zidx_v)
        pltpu.sync_copy(zval_hbm.at[wid], zval_v)
        pltpu.async_copy(zval_v, z_out.at[zidx_v], sem).wait()

    fn = _mpmd._mpmd_map(
        [(_sc_mesh(), body)],
        out_types=[
            jax.ShapeDtypeStruct((N, D), jnp.float32),
            jax.ShapeDtypeStruct(zmap.shape, zmap.dtype),
        ],
        input_output_aliases={2: 1},
        scratch_types=[
            pltpu.VMEM((per_w,), jnp.int32),
            pltpu.VMEM((per_w, D), jnp.float32),
            pltpu.VMEM((per_s,), jnp.int32),
            pltpu.VMEM((per_s,), jnp.int32),
            pltpu.SemaphoreType.DMA,
        ],
        compiler_params=pltpu.CompilerParams(use_tc_tiling_on_sc=True),
    )
    return fn(We, gidx, zmap, zidx2, zval2)


def _prologue(occT, w_sent, b_sent, bi_row, mi_row, bi_col, mi_col, M):
    U, L, NM = occT.shape
    B = _B

    def body(occ_ref, w_ref, b_ref, bi_ref, mi_ref, bic_ref, mic_ref,
             a_ref, zval_ref, zidx_ref):
        acc_logit = jnp.zeros((L, NM), jnp.float32)
        acc_sum = jnp.zeros((L, NM), jnp.float32)
        for u in range(U):
            s = occ_ref[u]
            acc_logit = acc_logit + s * w_ref[u]
            acc_sum = acc_sum + s
        likes = jax.nn.sigmoid(acc_logit + b_ref[0])
        mentioned = (acc_sum > 0).astype(jnp.float32)
        tri = (lax.broadcasted_iota(jnp.int32, (L, L), 1)
               <= lax.broadcasted_iota(jnp.int32, (L, L), 0)).astype(jnp.float32)
        cum = jnp.dot(tri, mentioned, preferred_element_type=jnp.float32)
        mask = (cum > 0.5).astype(jnp.float32)
        likes_m = likes * mask

        bi = bi_ref[:, :]
        mi = mi_ref[:, :]
        oh = (bi == lax.broadcasted_iota(jnp.int32, (B, NM), 0)
              ).astype(jnp.float32)
        a3 = likes_m[:, None, :] * oh[None, :, :]
        a_ref[:, :] = a3.reshape(L * B, NM)

        Pm = ((bic_ref[:, :] == bi)
              & (mic_ref[:, :] == mi)).astype(jnp.float32)
        ones_col = jnp.zeros((L, 1), jnp.float32) + 1.0
        s_col = lax.dot_general(mask, ones_col, (((0,), (0,)), ((), ())),
                                preferred_element_type=jnp.float32)
        zmax = jnp.max(Pm * s_col, axis=0, keepdims=True)
        zval_ref[:, :] = zmax.astype(jnp.int32)
        zidx_ref[:, :] = bi * M + mi

    return pl.pallas_call(
        body,
        out_shape=[
            jax.ShapeDtypeStruct((L * B, NM), jnp.float32),
            jax.ShapeDtypeStruct((1, NM), jnp.int32),
            jax.ShapeDtypeStruct((1, NM), jnp.int32),
        ],
        in_specs=[
            pl.BlockSpec(memory_space=pltpu.VMEM),
            pl.BlockSpec(memory_space=pltpu.SMEM),
            pl.BlockSpec(memory_space=pltpu.SMEM),
            pl.BlockSpec(memory_space=pltpu.VMEM),
            pl.BlockSpec(memory_space=pltpu.VMEM),
            pl.BlockSpec(memory_space=pltpu.VMEM),
            pl.BlockSpec(memory_space=pltpu.VMEM),
        ],
    )(occT, w_sent, b_sent, bi_row, mi_row, bi_col, mi_col)


def _encode(A, weg_pair, par_col, be_row):
    LB, NM = A.shape
    H = weg_pair.shape[1] // 2

    def body(a_ref, wp_ref, par_ref, be_ref, h_ref):
        sel = par_ref[:, :] == 1
        weg = jnp.where(sel, wp_ref[:, H:], wp_ref[:, :H])
        h = jnp.dot(a_ref[:, :], weg, preferred_element_type=jnp.float32)
        h_ref[:, :] = jax.nn.sigmoid(h + be_ref[:, :])

    return pl.pallas_call(
        body,
        out_shape=jax.ShapeDtypeStruct((LB, H), jnp.float32),
    )(A, weg_pair, par_col, be_row)


def _decoder_matmul(hs3, Wd, bd_row, z2d):
    B, L, H = hs3.shape
    M = Wd.shape[1]
    TB = 32
    TM = 5120
    grid_m = (M + TM - 1) // TM

    def body(hs_ref, wd_ref, bd_ref, z_ref, out_ref):
        wd = wd_ref[:, :]
        bdv = bd_ref[:, :]
        lio = lax.broadcasted_iota(jnp.int32, (L, TM), 0)
        for bb in range(TB):
            mm = jnp.dot(hs_ref[bb], wd,
                         preferred_element_type=jnp.float32)
            keep = ((lio + z_ref[bb:bb + 1, :]) < L).astype(jnp.float32)
            out_ref[bb] = (mm + bdv) * keep

    return pl.pallas_call(
        body,
        grid=(grid_m, B // TB),
        in_specs=[
            pl.BlockSpec((TB, L, H), lambda i, b: (b, 0, 0)),
            pl.BlockSpec((H, TM), lambda i, b: (0, i)),
            pl.BlockSpec((1, TM), lambda i, b: (0, i)),
            pl.BlockSpec((TB, TM), lambda i, b: (b, i)),
        ],
        out_specs=pl.BlockSpec((TB, L, TM), lambda i, b: (b, 0, i)),
        out_shape=jax.ShapeDtypeStruct((B, L, M), jnp.float32),
        compiler_params=pltpu.CompilerParams(
            dimension_semantics=("arbitrary", "arbitrary"),
        ),
    )(hs3, Wd, bd_row, z2d)


def kernel(movie_occurrences, batch_indices, movie_ids, w_sent, b_sent,
           We, be, Wd, bd):
    NM, L, U = movie_occurrences.shape
    M, H = We.shape
    B = _B

    occT = jnp.transpose(movie_occurrences, (2, 1, 0))
    bi_row = batch_indices.reshape(1, NM)
    mi_row = movie_ids.reshape(1, NM)

    A, zval, zidx = _prologue(
        occT, w_sent, b_sent, bi_row, mi_row,
        batch_indices.reshape(NM, 1), movie_ids.reshape(NM, 1), M)

    weg_pair, z1d = _sc_gather_scatter(
        We.reshape(M // 2, 2 * H), movie_ids // 2,
        jnp.zeros((B * M,), jnp.int32),
        zidx.reshape(_NW, NM // _NW), zval.reshape(_NW, NM // _NW))

    h_lb = _encode(A, weg_pair, (movie_ids % 2).reshape(NM, 1),
                   be.reshape(1, H))
    hs3 = h_lb.reshape(L, B, H).transpose(1, 0, 2)
    z2d = z1d.reshape(B, M)

    return _decoder_matmul(hs3, Wd, bd.reshape(1, M), z2d)


# speedup vs baseline: 1.0305x; 1.0194x over previous
import functools

import jax
import jax.numpy as jnp
from jax import lax
from jax.experimental import pallas as pl
from jax.experimental.pallas import tpu as pltpu
from jax.experimental.pallas import tpu_sc as plsc
from jax._src.pallas import mpmd as _mpmd

_B = 32
_NW = 32


def _sc_mesh():
    return plsc.VectorSubcoreMesh(core_axis_name="c", subcore_axis_name="s")


def _worker_id():
    return lax.axis_index("s") * 2 + lax.axis_index("c")


def _scatter_map(zmap, idx2, val2):
    per_w = idx2.shape[1]

    def body(z_in, idx_hbm, val_hbm, z_out, idx_v, val_v, sem):
        del z_in
        wid = _worker_id()
        pltpu.sync_copy(idx_hbm.at[wid], idx_v)
        pltpu.sync_copy(val_hbm.at[wid], val_v)
        pltpu.async_copy(val_v, z_out.at[idx_v], sem).wait()

    fn = _mpmd._mpmd_map(
        [(_sc_mesh(), body)],
        out_types=jax.ShapeDtypeStruct(zmap.shape, zmap.dtype),
        input_output_aliases={0: 0},
        scratch_types=[
            pltpu.VMEM((per_w,), jnp.int32),
            pltpu.VMEM((per_w,), jnp.int32),
            pltpu.SemaphoreType.DMA,
        ],
        compiler_params=pltpu.CompilerParams(use_tc_tiling_on_sc=False),
    )
    return fn(zmap, idx2, val2)


def _prologue(occT, w_sent, b_sent, bi_row, mi_row, bi_col, mi_col, M):
    U, L, NM = occT.shape
    B = _B

    def body(occ_ref, w_ref, b_ref, bi_ref, mi_ref, bic_ref, mic_ref,
             a_ref, zval_ref, zidx_ref):
        acc_logit = jnp.zeros((L, NM), jnp.float32)
        acc_sum = jnp.zeros((L, NM), jnp.float32)
        for u in range(U):
            s = occ_ref[u]
            acc_logit = acc_logit + s * w_ref[u]
            acc_sum = acc_sum + s
        likes = jax.nn.sigmoid(acc_logit + b_ref[0])
        mentioned = (acc_sum > 0).astype(jnp.float32)
        tri = (lax.broadcasted_iota(jnp.int32, (L, L), 1)
               <= lax.broadcasted_iota(jnp.int32, (L, L), 0)).astype(jnp.float32)
        cum = jnp.dot(tri, mentioned, preferred_element_type=jnp.float32)
        mask = (cum > 0.5).astype(jnp.float32)
        likes_m = likes * mask

        bi = bi_ref[:, :]
        mi = mi_ref[:, :]
        oh = (bi == lax.broadcasted_iota(jnp.int32, (B, NM), 0)
              ).astype(jnp.float32)
        a3 = likes_m[:, None, :] * oh[None, :, :]
        a_ref[:, :] = a3.reshape(L * B, NM)

        Pm = ((bic_ref[:, :] == bi)
              & (mic_ref[:, :] == mi)).astype(jnp.float32)
        ones_col = jnp.zeros((L, 1), jnp.float32) + 1.0
        s_col = lax.dot_general(mask, ones_col, (((0,), (0,)), ((), ())),
                                preferred_element_type=jnp.float32)
        zmax = jnp.max(Pm * s_col, axis=0, keepdims=True)
        zval_ref[:, :] = zmax.astype(jnp.int32)
        zidx_ref[:, :] = bi * M + mi

    return pl.pallas_call(
        body,
        out_shape=[
            jax.ShapeDtypeStruct((L * B, NM), jnp.float32),
            jax.ShapeDtypeStruct((1, NM), jnp.int32),
            jax.ShapeDtypeStruct((1, NM), jnp.int32),
        ],
        in_specs=[
            pl.BlockSpec(memory_space=pltpu.VMEM),
            pl.BlockSpec(memory_space=pltpu.SMEM),
            pl.BlockSpec(memory_space=pltpu.SMEM),
            pl.BlockSpec(memory_space=pltpu.VMEM),
            pl.BlockSpec(memory_space=pltpu.VMEM),
            pl.BlockSpec(memory_space=pltpu.VMEM),
            pl.BlockSpec(memory_space=pltpu.VMEM),
        ],
    )(occT, w_sent, b_sent, bi_row, mi_row, bi_col, mi_col)


def _encode(A, We, mi_row, be_row):
    LB, NM = A.shape
    H = We.shape[1]
    WAVE = 32

    def body(a_ref, we_hbm, mi_ref, be_ref, h_ref, weg_v, sem):
        for w in range(NM // WAVE):
            copies = []
            for j in range(WAVE):
                i = w * WAVE + j
                c = pltpu.make_async_copy(
                    we_hbm.at[pl.ds(mi_ref[0, i], 1), :],
                    weg_v.at[pl.ds(i, 1), :], sem)
                c.start()
                copies.append(c)
            for c in copies:
                c.wait()
        h = jnp.dot(a_ref[:, :], weg_v[:, :],
                    preferred_element_type=jnp.float32)
        h_ref[:, :] = jax.nn.sigmoid(h + be_ref[:, :])

    return pl.pallas_call(
        body,
        out_shape=jax.ShapeDtypeStruct((LB, H), jnp.float32),
        in_specs=[
            pl.BlockSpec(memory_space=pltpu.VMEM),
            pl.BlockSpec(memory_space=pltpu.HBM),
            pl.BlockSpec(memory_space=pltpu.SMEM),
            pl.BlockSpec(memory_space=pltpu.VMEM),
        ],
        scratch_shapes=[
            pltpu.VMEM((NM, H), jnp.float32),
            pltpu.SemaphoreType.DMA,
        ],
    )(A, We, mi_row, be_row)


def _decoder_matmul(hs3, Wd, bd_row, z2d):
    B, L, H = hs3.shape
    M = Wd.shape[1]
    TB = 32
    TM = 5120
    grid_m = (M + TM - 1) // TM

    def body(hs_ref, wd_ref, bd_ref, z_ref, out_ref):
        wd = wd_ref[:, :]
        bdv = bd_ref[:, :]
        lio = lax.broadcasted_iota(jnp.int32, (L, TM), 0)
        for bb in range(TB):
            mm = jnp.dot(hs_ref[bb], wd,
                         preferred_element_type=jnp.float32)
            keep = ((lio + z_ref[bb:bb + 1, :]) < L).astype(jnp.float32)
            out_ref[bb] = (mm + bdv) * keep

    return pl.pallas_call(
        body,
        grid=(grid_m, B // TB),
        in_specs=[
            pl.BlockSpec((TB, L, H), lambda i, b: (b, 0, 0)),
            pl.BlockSpec((H, TM), lambda i, b: (0, i)),
            pl.BlockSpec((1, TM), lambda i, b: (0, i)),
            pl.BlockSpec((TB, TM), lambda i, b: (b, i)),
        ],
        out_specs=pl.BlockSpec((TB, L, TM), lambda i, b: (b, 0, i)),
        out_shape=jax.ShapeDtypeStruct((B, L, M), jnp.float32),
        compiler_params=pltpu.CompilerParams(
            dimension_semantics=("arbitrary", "arbitrary"),
        ),
    )(hs3, Wd, bd_row, z2d)


def kernel(movie_occurrences, batch_indices, movie_ids, w_sent, b_sent,
           We, be, Wd, bd):
    NM, L, U = movie_occurrences.shape
    M, H = We.shape
    B = _B

    occT = jnp.transpose(movie_occurrences, (2, 1, 0))
    bi_row = batch_indices.reshape(1, NM)
    mi_row = movie_ids.reshape(1, NM)

    A, zval, zidx = _prologue(
        occT, w_sent, b_sent, bi_row, mi_row,
        batch_indices.reshape(NM, 1), movie_ids.reshape(NM, 1), M)

    z1d = _scatter_map(jnp.zeros((B * M,), jnp.int32),
                       zidx.reshape(_NW, NM // _NW),
                       zval.reshape(_NW, NM // _NW))

    h_lb = _encode(A, We, mi_row, be.reshape(1, H))
    hs3 = h_lb.reshape(L, B, H).transpose(1, 0, 2)
    z2d = z1d.reshape(B, M)

    return _decoder_matmul(hs3, Wd, bd.reshape(1, M), z2d)
